# baseline (speedup 1.0000x reference)
import functools

import jax
import jax.numpy as jnp
from jax import lax
from jax.experimental import pallas as pl
from jax.experimental.pallas import tpu as pltpu
from jax.experimental.pallas import tpu_sc as plsc

WIN = 500
K = 2
TC_WB = 32
TC_SHARE2 = 64
SC_WG = 8
SC_CHUNKS = 33
SC_SPAN = 1152

NEG_INF = float("-inf")


def _tc_one_panel(mx, r_ref, p_ref, idx_ref, wt0, wt1):
    inv = 1.0 / WIN
    r = r_ref[...]
    ms = []
    for j in range(TC_WB):
        a = lax.slice(mx, (0, j * WIN), (8, (j + 1) * WIN))
        b = lax.slice(r, (0, j * WIN), (16, (j + 1) * WIN))
        m = lax.dot_general(
            a, b,
            dimension_numbers=(((1,), (1,)), ((), ())),
            preferred_element_type=jnp.float32,
            precision=lax.Precision.HIGHEST,
        )
        ms.append(m * inv)
    M = jnp.stack(ms, axis=0)
    li = lax.broadcasted_iota(jnp.int32, M.shape, 2)
    max1 = jnp.max(M, axis=-1)
    i1 = jnp.min(jnp.where(M == max1[..., None], li, 127), axis=-1)
    M2 = jnp.where(li == i1[..., None], NEG_INF, M)
    max2 = jnp.max(M2, axis=-1)
    i2 = jnp.min(jnp.where(M2 == max2[..., None], li, 127), axis=-1)
    p_ref[...] = max1 * wt0 + max2 * wt1
    idx_ref[...] = jnp.stack([i1, i2], axis=1)


def _tc_fused_body(wt_ref, mx_ref, ra_ref, rb_ref, rc_ref,
                   pa_ref, ia_ref, pb_ref, ib_ref, pc_ref, ic_ref):
    mx = mx_ref[...]
    wt0 = wt_ref[0, 0]
    wt1 = wt_ref[1, 0]
    _tc_one_panel(mx, ra_ref, pa_ref, ia_ref, wt0, wt1)
    _tc_one_panel(mx, rb_ref, pb_ref, ib_ref, wt0, wt1)

    @pl.when(pl.program_id(0) < TC_SHARE2 // TC_WB)
    def _():
        _tc_one_panel(mx, rc_ref, pc_ref, ic_ref, wt0, wt1)


def _tc_panels(mixed, ref_a, ref_b, ref_c, weights, bs, n_refs, n_windows):
    grid = -(-n_windows // TC_WB)
    nw_pad = grid * TC_WB
    cols = TC_WB * WIN
    nshare = TC_SHARE2 // TC_WB
    out_shape = [
        jax.ShapeDtypeStruct((nw_pad, bs), jnp.float32),
        jax.ShapeDtypeStruct((nw_pad, K, bs), jnp.int32),
    ] * 3
    in_specs = [
        pl.BlockSpec(memory_space=pltpu.SMEM),
        pl.BlockSpec((bs, cols), lambda i: (0, i)),
        pl.BlockSpec((n_refs, cols), lambda i: (0, i)),
        pl.BlockSpec((n_refs, cols), lambda i: (0, i)),
        pl.BlockSpec((n_refs, cols), lambda i: (0, jnp.minimum(i, nshare - 1))),
    ]
    out_specs = [
        pl.BlockSpec((TC_WB, bs), lambda i: (i, 0)),
        pl.BlockSpec((TC_WB, K, bs), lambda i: (i, 0, 0)),
    ] * 3
    pa, ia, pb, ib, pc, ic = pl.pallas_call(
        _tc_fused_body,
        grid=(grid,),
        in_specs=in_specs,
        out_specs=out_specs,
        out_shape=out_shape,
    )(weights, mixed, ref_a, ref_b, ref_c)
    outs = []
    for p_t, i_t in ((pa, ia), (pb, ib)):
        outs.append(p_t[:n_windows].T)
        outs.append(jnp.transpose(i_t[:n_windows], (2, 1, 0)))
    outs.append(pc[:TC_SHARE2])
    outs.append(ic[:TC_SHARE2])
    return outs


def _sc_dense_body(nw_sc, wbase, mx_hbm, r_hbm, wts, po, i1o, i2o,
                   mxbuf, refbuf, accbuf, wtbl, wtbuf, opbuf, ob1buf, ob2buf):
    n_tasks = nw_sc // SC_WG
    cid = lax.axis_index("c")
    sid = lax.axis_index("s")
    wid = sid * 2 + cid

    @pl.when(wid < n_tasks)
    def _():
        pltpu.sync_copy(wts, wtbuf)
        t0 = pl.multiple_of(wid * SC_WG, 8)
        ridx = lax.iota(jnp.int32, 16)
        lanes = lax.iota(jnp.int32, 16)

        def pair_step(pair, carry):
            col0 = (wbase + t0 + 2 * pair) * WIN
            c_lo = pl.multiple_of((col0 // 128) * 128, 128)
            delta = pl.multiple_of(col0 - c_lo, 8)
            pltpu.sync_copy(mx_hbm.at[:, pl.ds(c_lo, SC_SPAN)], mxbuf)
            pltpu.sync_copy(r_hbm.at[:, pl.ds(c_lo, SC_SPAN)], refbuf)
            for bb in range(4):
                b0, b1 = 2 * bb, 2 * bb + 1
                for half in range(2):
                    lo = delta + half * WIN
                    hi = lo + WIN
                    off0 = pl.multiple_of((lo // 16) * 16, 16)

                    def chunk_step(i, accs):
                        base = off0 + i * 16
                        pos = base + lanes
                        maskf = jnp.where(
                            (pos >= lo) & (pos < hi), 1.0, 0.0
                        ).astype(jnp.float32)
                        m0 = mxbuf[b0, pl.ds(base, 16)] * maskf
                        m1 = mxbuf[b1, pl.ds(base, 16)] * maskf
                        acc0 = list(accs[:16])
                        acc1 = list(accs[16:])
                        for rr in range(16):
                            rc = refbuf[rr, pl.ds(base, 16)]
                            acc0[rr] = acc0[rr] + m0 * rc
                            acc1[rr] = acc1[rr] + m1 * rc
                        return tuple(acc0) + tuple(acc1)

                    init = (jnp.zeros((16,), jnp.float32),) * 32
                    accs = lax.fori_loop(0, SC_CHUNKS, chunk_step, init,
                                         unroll=3)
                    for j in range(2):
                        for rr in range(16):
                            accbuf[j, rr, :] = accs[16 * j + rr]
                    w = 2 * pair + half
                    for j, b in ((0, b0), (1, b1)):
                        vsum = jnp.zeros((16,), jnp.float32)
                        for l in range(16):
                            col = plsc.load_gather(
                                accbuf.at[j],
                                [ridx, jnp.full((16,), l, jnp.int32)],
                            )
                            vsum = vsum + col
                        wtbl[b, w, :] = vsum * (1.0 / WIN)
            return carry

        lax.fori_loop(0, SC_WG // 2, pair_step, 0)

        wt0 = wtbuf[0, :]
        wt1 = wtbuf[1, :]
        widx = lax.iota(jnp.int32, 16)
        neg = jnp.full((16,), NEG_INF, jnp.float32)
        zero_i = jnp.zeros((16,), jnp.int32)
        for b in range(8):
            best = plsc.load_gather(wtbl.at[b], [widx, zero_i])
            bidx = zero_i
            sec = neg
            sidx = zero_i
            for rr in range(1, 16):
                v = plsc.load_gather(
                    wtbl.at[b], [widx, jnp.full((16,), rr, jnp.int32)]
                )
                rvec = jnp.full((16,), rr, jnp.int32)
                c1 = v > best
                c2 = v > sec
                sec = jnp.where(c1, best, jnp.where(c2, v, sec))
                sidx = jnp.where(c1, bidx, jnp.where(c2, rvec, sidx))
                best = jnp.where(c1, v, best)
                bidx = jnp.where(c1, rvec, bidx)
            bvec = jnp.full((16,), b, jnp.int32)
            plsc.store_scatter(opbuf, [widx, bvec], best * wt0 + sec * wt1)
            plsc.store_scatter(ob1buf, [widx, bvec], bidx)
            plsc.store_scatter(ob2buf, [widx, bvec], sidx)

        pltpu.sync_copy(opbuf.at[pl.ds(0, SC_WG), :],
                        po.at[pl.ds(t0, SC_WG), :])
        pltpu.sync_copy(ob1buf.at[pl.ds(0, SC_WG), :],
                        i1o.at[pl.ds(t0, SC_WG), :])
        pltpu.sync_copy(ob2buf.at[pl.ds(0, SC_WG), :],
                        i2o.at[pl.ds(t0, SC_WG), :])


def _sc_panel(mixed, ref, weights, bs, n_windows, wbase=0):
    nw_sc = n_windows - wbase
    mesh = plsc.VectorSubcoreMesh(
        core_axis_name="c", subcore_axis_name="s", num_cores=2, num_subcores=16
    )
    out_type = [
        jax.ShapeDtypeStruct((nw_sc, bs), jnp.float32),
        jax.ShapeDtypeStruct((nw_sc, bs), jnp.int32),
        jax.ShapeDtypeStruct((nw_sc, bs), jnp.int32),
    ]
    scratch = [
        pltpu.VMEM((8, SC_SPAN), jnp.float32),
        pltpu.VMEM((16, SC_SPAN), jnp.float32),
        pltpu.VMEM((2, 16, 16), jnp.float32),
        pltpu.VMEM((8, 16, 16), jnp.float32),
        pltpu.VMEM((K, 16), jnp.float32),
        pltpu.VMEM((16, 8), jnp.float32),
        pltpu.VMEM((16, 8), jnp.int32),
        pltpu.VMEM((16, 8), jnp.int32),
    ]
    body = functools.partial(_sc_dense_body, nw_sc, wbase)
    fn = pl.kernel(
        body,
        out_type=out_type,
        mesh=mesh,
        scratch_types=scratch,
        compiler_params=pltpu.CompilerParams(
            needs_layout_passes=False, use_tc_tiling_on_sc=True
        ),
    )
    wts16 = jnp.broadcast_to(weights[:K], (K, 16))
    return fn(mixed, ref, wts16)


def kernel(input_mixed, ref_panel_0, ref_panel_1, ref_panel_2, weights):
    bs, n_snps = input_mixed.shape
    n_refs = ref_panel_0.shape[0]
    n_windows = n_snps // WIN
    p0, i0, p1, i1, pc_t, ic_t = _tc_panels(
        input_mixed, ref_panel_0, ref_panel_1, ref_panel_2, weights,
        bs, n_refs, n_windows
    )
    ps_t, is1_t, is2_t = _sc_panel(
        input_mixed, ref_panel_2, weights, bs, n_windows, wbase=TC_SHARE2
    )
    p2 = jnp.concatenate([pc_t, ps_t], axis=0).T
    i_sc = jnp.stack([is1_t, is2_t], axis=1)
    i2 = jnp.transpose(jnp.concatenate([ic_t, i_sc], axis=0), (2, 1, 0))
    return (p0, p1, p2, i0, i1, i2)


# speedup vs baseline: 1.2300x; 1.2300x over previous
import functools

import jax
import jax.numpy as jnp
from jax import lax
from jax.experimental import pallas as pl
from jax.experimental.pallas import tpu as pltpu
from jax.experimental.pallas import tpu_sc as plsc

WIN = 500
K = 2
TC_WB = 32
SC_WG = 8
SC_CHUNKS = 33
SC_SPAN = 1152

NEG_INF = float("-inf")


def _tc_fused_body(wt_ref, mx_ref, ra_ref, rb_ref,
                   pa_ref, ia_ref, pb_ref, ib_ref):
    mx = mx_ref[...]
    inv = 1.0 / WIN
    wt0 = wt_ref[0, 0]
    wt1 = wt_ref[1, 0]
    for r_ref, p_ref, idx_ref in ((ra_ref, pa_ref, ia_ref),
                                  (rb_ref, pb_ref, ib_ref)):
        r = r_ref[...]
        ms = []
        for j in range(TC_WB):
            a = lax.slice(mx, (0, j * WIN), (8, (j + 1) * WIN))
            b = lax.slice(r, (0, j * WIN), (16, (j + 1) * WIN))
            m = lax.dot_general(
                a, b,
                dimension_numbers=(((1,), (1,)), ((), ())),
                preferred_element_type=jnp.float32,
                precision=lax.Precision.HIGHEST,
            )
            ms.append(m * inv)
        M = jnp.stack(ms, axis=0)
        li = lax.broadcasted_iota(jnp.int32, M.shape, 2)
        max1 = jnp.max(M, axis=-1)
        i1 = jnp.min(jnp.where(M == max1[..., None], li, 127), axis=-1)
        M2 = jnp.where(li == i1[..., None], NEG_INF, M)
        max2 = jnp.max(M2, axis=-1)
        i2 = jnp.min(jnp.where(M2 == max2[..., None], li, 127), axis=-1)
        p_ref[...] = max1 * wt0 + max2 * wt1
        idx_ref[...] = jnp.stack([i1, i2], axis=1)


def _tc_panels(mixed, ref_a, ref_b, weights, bs, n_refs, n_windows):
    grid = -(-n_windows // TC_WB)
    nw_pad = grid * TC_WB
    cols = TC_WB * WIN
    out_shape = [
        jax.ShapeDtypeStruct((nw_pad, bs), jnp.float32),
        jax.ShapeDtypeStruct((nw_pad, K, bs), jnp.int32),
    ] * 2
    in_specs = [
        pl.BlockSpec(memory_space=pltpu.SMEM),
        pl.BlockSpec((bs, cols), lambda i: (0, i)),
        pl.BlockSpec((n_refs, cols), lambda i: (0, i)),
        pl.BlockSpec((n_refs, cols), lambda i: (0, i)),
    ]
    out_specs = [
        pl.BlockSpec((TC_WB, bs), lambda i: (i, 0)),
        pl.BlockSpec((TC_WB, K, bs), lambda i: (i, 0, 0)),
    ] * 2
    pa, ia, pb, ib = pl.pallas_call(
        _tc_fused_body,
        grid=(grid,),
        in_specs=in_specs,
        out_specs=out_specs,
        out_shape=out_shape,
    )(weights, mixed, ref_a, ref_b)
    outs = []
    for p_t, i_t in ((pa, ia), (pb, ib)):
        outs.append(p_t[:n_windows].T)
        outs.append(jnp.transpose(i_t[:n_windows], (2, 1, 0)))
    return outs


def _sc_dense_body(nw, mx_hbm, r_hbm, wts, po, i1o, i2o,
                   mxbuf, refbuf, accbuf, wtbl, wtbuf, opbuf, ob1buf, ob2buf):
    n_tasks = nw // SC_WG
    cid = lax.axis_index("c")
    sid = lax.axis_index("s")
    wid = sid * 2 + cid

    @pl.when(wid < n_tasks)
    def _():
        pltpu.sync_copy(wts, wtbuf)
        t0 = pl.multiple_of(wid * SC_WG, 8)
        ridx = lax.iota(jnp.int32, 16)
        lanes = lax.iota(jnp.int32, 16)

        def pair_step(pair, carry):
            col0 = (t0 + 2 * pair) * WIN
            c_lo = pl.multiple_of((col0 // 128) * 128, 128)
            delta = pl.multiple_of(col0 - c_lo, 8)
            pltpu.sync_copy(mx_hbm.at[:, pl.ds(c_lo, SC_SPAN)], mxbuf)
            pltpu.sync_copy(r_hbm.at[:, pl.ds(c_lo, SC_SPAN)], refbuf)
            for half in range(2):
                lo = delta + half * WIN
                hi = lo + WIN
                off0 = pl.multiple_of((lo // 16) * 16, 16)
                for rb in range(4):

                    def chunk_step(i, accs):
                        base = off0 + i * 16
                        pos = base + lanes
                        maskf = jnp.where(
                            (pos >= lo) & (pos < hi), 1.0, 0.0
                        ).astype(jnp.float32)
                        rcs = [refbuf[4 * rb + q, pl.ds(base, 16)] * maskf
                               for q in range(4)]
                        out = list(accs)
                        for b in range(8):
                            mxv = mxbuf[b, pl.ds(base, 16)]
                            for q in range(4):
                                out[4 * b + q] = out[4 * b + q] + mxv * rcs[q]
                        return tuple(out)

                    init = (jnp.zeros((16,), jnp.float32),) * 32
                    accs = lax.fori_loop(0, SC_CHUNKS, chunk_step, init)
                    for b in range(8):
                        for q in range(4):
                            accbuf[b, 4 * rb + q, :] = accs[4 * b + q]
                w = 2 * pair + half
                for b in range(8):
                    vsum = jnp.zeros((16,), jnp.float32)
                    for l in range(16):
                        col = plsc.load_gather(
                            accbuf.at[b],
                            [ridx, jnp.full((16,), l, jnp.int32)],
                        )
                        vsum = vsum + col
                    wtbl[b, w, :] = vsum * (1.0 / WIN)
            return carry

        lax.fori_loop(0, SC_WG // 2, pair_step, 0)

        wt0 = wtbuf[0, :]
        wt1 = wtbuf[1, :]
        widx = lax.iota(jnp.int32, 16)
        neg = jnp.full((16,), NEG_INF, jnp.float32)
        zero_i = jnp.zeros((16,), jnp.int32)
        for b in range(8):
            best = plsc.load_gather(wtbl.at[b], [widx, zero_i])
            bidx = zero_i
            sec = neg
            sidx = zero_i
            for rr in range(1, 16):
                v = plsc.load_gather(
                    wtbl.at[b], [widx, jnp.full((16,), rr, jnp.int32)]
                )
                rvec = jnp.full((16,), rr, jnp.int32)
                c1 = v > best
                c2 = v > sec
                sec = jnp.where(c1, best, jnp.where(c2, v, sec))
                sidx = jnp.where(c1, bidx, jnp.where(c2, rvec, sidx))
                best = jnp.where(c1, v, best)
                bidx = jnp.where(c1, rvec, bidx)
            bvec = jnp.full((16,), b, jnp.int32)
            plsc.store_scatter(opbuf, [widx, bvec], best * wt0 + sec * wt1)
            plsc.store_scatter(ob1buf, [widx, bvec], bidx)
            plsc.store_scatter(ob2buf, [widx, bvec], sidx)

        pltpu.sync_copy(opbuf.at[pl.ds(0, SC_WG), :],
                        po.at[pl.ds(t0, SC_WG), :])
        pltpu.sync_copy(ob1buf.at[pl.ds(0, SC_WG), :],
                        i1o.at[pl.ds(t0, SC_WG), :])
        pltpu.sync_copy(ob2buf.at[pl.ds(0, SC_WG), :],
                        i2o.at[pl.ds(t0, SC_WG), :])


def _sc_panel(mixed, ref, weights, bs, n_windows):
    mesh = plsc.VectorSubcoreMesh(
        core_axis_name="c", subcore_axis_name="s", num_cores=2, num_subcores=16
    )
    out_type = [
        jax.ShapeDtypeStruct((n_windows, bs), jnp.float32),
        jax.ShapeDtypeStruct((n_windows, bs), jnp.int32),
        jax.ShapeDtypeStruct((n_windows, bs), jnp.int32),
    ]
    scratch = [
        pltpu.VMEM((8, SC_SPAN), jnp.float32),
        pltpu.VMEM((16, SC_SPAN), jnp.float32),
        pltpu.VMEM((8, 16, 16), jnp.float32),
        pltpu.VMEM((8, 16, 16), jnp.float32),
        pltpu.VMEM((K, 16), jnp.float32),
        pltpu.VMEM((16, 8), jnp.float32),
        pltpu.VMEM((16, 8), jnp.int32),
        pltpu.VMEM((16, 8), jnp.int32),
    ]
    body = functools.partial(_sc_dense_body, n_windows)
    fn = pl.kernel(
        body,
        out_type=out_type,
        mesh=mesh,
        scratch_types=scratch,
        compiler_params=pltpu.CompilerParams(
            needs_layout_passes=False, use_tc_tiling_on_sc=True
        ),
    )
    wts16 = jnp.broadcast_to(weights[:K], (K, 16))
    p_t, i1_t, i2_t = fn(mixed, ref, wts16)
    pooled = p_t.T
    idx = jnp.stack([i1_t.T, i2_t.T], axis=1)
    return pooled, idx


def kernel(input_mixed, ref_panel_0, ref_panel_1, ref_panel_2, weights):
    bs, n_snps = input_mixed.shape
    n_refs = ref_panel_0.shape[0]
    n_windows = n_snps // WIN
    p0, i0, p1, i1 = _tc_panels(
        input_mixed, ref_panel_0, ref_panel_1, weights, bs, n_refs, n_windows
    )
    p2, i2 = _sc_panel(input_mixed, ref_panel_2, weights, bs, n_windows)
    return (p0, p1, p2, i0, i1, i2)


# speedup vs baseline: 1.3561x; 1.1026x over previous
import functools

import jax
import jax.numpy as jnp
from jax import lax
from jax.experimental import pallas as pl
from jax.experimental.pallas import tpu as pltpu
from jax.experimental.pallas import tpu_sc as plsc

WIN = 500
K = 2
TC_WB = 32
SC_WG = 8
SC_CHUNKS = 33
SC_SPAN = 1152

NEG_INF = float("-inf")


def _tc_fused_body(wt_ref, mx_ref, ra_ref, rb_ref,
                   pa_ref, ia_ref, pb_ref, ib_ref):
    mx = mx_ref[...]
    inv = 1.0 / WIN
    wt0 = wt_ref[0, 0]
    wt1 = wt_ref[1, 0]
    for r_ref, p_ref, idx_ref in ((ra_ref, pa_ref, ia_ref),
                                  (rb_ref, pb_ref, ib_ref)):
        r = r_ref[...]
        ms = []
        for j in range(TC_WB):
            a = lax.slice(mx, (0, j * WIN), (8, (j + 1) * WIN))
            b = lax.slice(r, (0, j * WIN), (16, (j + 1) * WIN))
            m = lax.dot_general(
                a, b,
                dimension_numbers=(((1,), (1,)), ((), ())),
                preferred_element_type=jnp.float32,
                precision=lax.Precision.HIGHEST,
            )
            ms.append(m * inv)
        M = jnp.stack(ms, axis=0)
        li = lax.broadcasted_iota(jnp.int32, M.shape, 2)
        max1 = jnp.max(M, axis=-1)
        i1 = jnp.min(jnp.where(M == max1[..., None], li, 127), axis=-1)
        M2 = jnp.where(li == i1[..., None], NEG_INF, M)
        max2 = jnp.max(M2, axis=-1)
        i2 = jnp.min(jnp.where(M2 == max2[..., None], li, 127), axis=-1)
        p_ref[...] = max1 * wt0 + max2 * wt1
        idx_ref[...] = jnp.stack([i1, i2], axis=1)


def _tc_panels(mixed, ref_a, ref_b, weights, bs, n_refs, n_windows):
    grid = -(-n_windows // TC_WB)
    nw_pad = grid * TC_WB
    cols = TC_WB * WIN
    out_shape = [
        jax.ShapeDtypeStruct((nw_pad, bs), jnp.float32),
        jax.ShapeDtypeStruct((nw_pad, K, bs), jnp.int32),
    ] * 2
    in_specs = [
        pl.BlockSpec(memory_space=pltpu.SMEM),
        pl.BlockSpec((bs, cols), lambda i: (0, i)),
        pl.BlockSpec((n_refs, cols), lambda i: (0, i)),
        pl.BlockSpec((n_refs, cols), lambda i: (0, i)),
    ]
    out_specs = [
        pl.BlockSpec((TC_WB, bs), lambda i: (i, 0)),
        pl.BlockSpec((TC_WB, K, bs), lambda i: (i, 0, 0)),
    ] * 2
    pa, ia, pb, ib = pl.pallas_call(
        _tc_fused_body,
        grid=(grid,),
        in_specs=in_specs,
        out_specs=out_specs,
        out_shape=out_shape,
    )(weights, mixed, ref_a, ref_b)
    outs = []
    for p_t, i_t in ((pa, ia), (pb, ib)):
        outs.append(p_t[:n_windows].T)
        outs.append(jnp.transpose(i_t[:n_windows], (2, 1, 0)))
    return outs


def _sc_dense_body(nw, mx_hbm, r_hbm, wts, po, i1o, i2o,
                   mxbuf, refbuf, accbuf, wtbl, wtbuf, opbuf, ob1buf, ob2buf,
                   sem0, sem1):
    n_tasks = nw // SC_WG
    cid = lax.axis_index("c")
    sid = lax.axis_index("s")
    wid = sid * 2 + cid

    @pl.when(wid < n_tasks)
    def _():
        pltpu.sync_copy(wts, wtbuf)
        t0 = pl.multiple_of(wid * SC_WG, 8)
        ridx = lax.iota(jnp.int32, 16)
        lanes = lax.iota(jnp.int32, 16)

        n_pairs = SC_WG // 2

        def _pair_cols(p):
            col0 = (t0 + 2 * p) * WIN
            c_lo = pl.multiple_of((col0 // 128) * 128, 128)
            return col0, c_lo

        _, c0_lo = _pair_cols(0)
        pltpu.async_copy(mx_hbm.at[:, pl.ds(c0_lo, SC_SPAN)],
                         mxbuf.at[0], sem0)
        pltpu.async_copy(r_hbm.at[:, pl.ds(c0_lo, SC_SPAN)],
                         refbuf.at[0], sem0)

        def pair_step(pair, carry):
            col0, c_lo = _pair_cols(pair)
            delta = pl.multiple_of(col0 - c_lo, 8)
            par = lax.rem(pair, 2)
            _, cn_lo = _pair_cols(pair + 1)

            @pl.when((pair + 1 < n_pairs) & (par == 0))
            def _():
                pltpu.async_copy(mx_hbm.at[:, pl.ds(cn_lo, SC_SPAN)],
                                 mxbuf.at[1], sem1)
                pltpu.async_copy(r_hbm.at[:, pl.ds(cn_lo, SC_SPAN)],
                                 refbuf.at[1], sem1)

            @pl.when((pair + 1 < n_pairs) & (par == 1))
            def _():
                pltpu.async_copy(mx_hbm.at[:, pl.ds(cn_lo, SC_SPAN)],
                                 mxbuf.at[0], sem0)
                pltpu.async_copy(r_hbm.at[:, pl.ds(cn_lo, SC_SPAN)],
                                 refbuf.at[0], sem0)

            @pl.when(par == 0)
            def _():
                pltpu.make_async_copy(mx_hbm.at[:, pl.ds(c_lo, SC_SPAN)],
                                      mxbuf.at[0], sem0).wait()
                pltpu.make_async_copy(r_hbm.at[:, pl.ds(c_lo, SC_SPAN)],
                                      refbuf.at[0], sem0).wait()

            @pl.when(par == 1)
            def _():
                pltpu.make_async_copy(mx_hbm.at[:, pl.ds(c_lo, SC_SPAN)],
                                      mxbuf.at[1], sem1).wait()
                pltpu.make_async_copy(r_hbm.at[:, pl.ds(c_lo, SC_SPAN)],
                                      refbuf.at[1], sem1).wait()

            for half in range(2):
                lo = delta + half * WIN
                hi = lo + WIN
                off0 = pl.multiple_of((lo // 16) * 16, 16)
                for rb in range(4):

                    def chunk_step(i, accs):
                        base = off0 + i * 16
                        pos = base + lanes
                        maskf = jnp.where(
                            (pos >= lo) & (pos < hi), 1.0, 0.0
                        ).astype(jnp.float32)
                        rcs = [refbuf[par, 4 * rb + q, pl.ds(base, 16)] * maskf
                               for q in range(4)]
                        out = list(accs)
                        for b in range(8):
                            mxv = mxbuf[par, b, pl.ds(base, 16)]
                            for q in range(4):
                                out[4 * b + q] = out[4 * b + q] + mxv * rcs[q]
                        return tuple(out)

                    init = (jnp.zeros((16,), jnp.float32),) * 32
                    accs = lax.fori_loop(0, SC_CHUNKS, chunk_step, init)
                    for b in range(8):
                        for q in range(4):
                            accbuf[b, 4 * rb + q, :] = accs[4 * b + q]
                w = 2 * pair + half
                for b in range(8):
                    vsum = jnp.zeros((16,), jnp.float32)
                    for l in range(16):
                        col = plsc.load_gather(
                            accbuf.at[b],
                            [ridx, jnp.full((16,), l, jnp.int32)],
                        )
                        vsum = vsum + col
                    wtbl[b, w, :] = vsum * (1.0 / WIN)
            return carry

        lax.fori_loop(0, SC_WG // 2, pair_step, 0)

        wt0 = wtbuf[0, :]
        wt1 = wtbuf[1, :]
        widx = lax.iota(jnp.int32, 16)
        neg = jnp.full((16,), NEG_INF, jnp.float32)
        zero_i = jnp.zeros((16,), jnp.int32)
        for b in range(8):
            best = plsc.load_gather(wtbl.at[b], [widx, zero_i])
            bidx = zero_i
            sec = neg
            sidx = zero_i
            for rr in range(1, 16):
                v = plsc.load_gather(
                    wtbl.at[b], [widx, jnp.full((16,), rr, jnp.int32)]
                )
                rvec = jnp.full((16,), rr, jnp.int32)
                c1 = v > best
                c2 = v > sec
                sec = jnp.where(c1, best, jnp.where(c2, v, sec))
                sidx = jnp.where(c1, bidx, jnp.where(c2, rvec, sidx))
                best = jnp.where(c1, v, best)
                bidx = jnp.where(c1, rvec, bidx)
            bvec = jnp.full((16,), b, jnp.int32)
            plsc.store_scatter(opbuf, [widx, bvec], best * wt0 + sec * wt1)
            plsc.store_scatter(ob1buf, [widx, bvec], bidx)
            plsc.store_scatter(ob2buf, [widx, bvec], sidx)

        pltpu.sync_copy(opbuf.at[pl.ds(0, SC_WG), :],
                        po.at[pl.ds(t0, SC_WG), :])
        pltpu.sync_copy(ob1buf.at[pl.ds(0, SC_WG), :],
                        i1o.at[pl.ds(t0, SC_WG), :])
        pltpu.sync_copy(ob2buf.at[pl.ds(0, SC_WG), :],
                        i2o.at[pl.ds(t0, SC_WG), :])


def _sc_panel(mixed, ref, weights, bs, n_windows):
    mesh = plsc.VectorSubcoreMesh(
        core_axis_name="c", subcore_axis_name="s", num_cores=2, num_subcores=16
    )
    out_type = [
        jax.ShapeDtypeStruct((n_windows, bs), jnp.float32),
        jax.ShapeDtypeStruct((n_windows, bs), jnp.int32),
        jax.ShapeDtypeStruct((n_windows, bs), jnp.int32),
    ]
    scratch = [
        pltpu.VMEM((2, 8, SC_SPAN), jnp.float32),
        pltpu.VMEM((2, 16, SC_SPAN), jnp.float32),
        pltpu.VMEM((8, 16, 16), jnp.float32),
        pltpu.VMEM((8, 16, 16), jnp.float32),
        pltpu.VMEM((K, 16), jnp.float32),
        pltpu.VMEM((16, 8), jnp.float32),
        pltpu.VMEM((16, 8), jnp.int32),
        pltpu.VMEM((16, 8), jnp.int32),
        pltpu.SemaphoreType.DMA,
        pltpu.SemaphoreType.DMA,
    ]
    body = functools.partial(_sc_dense_body, n_windows)
    fn = pl.kernel(
        body,
        out_type=out_type,
        mesh=mesh,
        scratch_types=scratch,
        compiler_params=pltpu.CompilerParams(
            needs_layout_passes=False, use_tc_tiling_on_sc=True
        ),
    )
    wts16 = jnp.broadcast_to(weights[:K], (K, 16))
    p_t, i1_t, i2_t = fn(mixed, ref, wts16)
    pooled = p_t.T
    idx = jnp.stack([i1_t.T, i2_t.T], axis=1)
    return pooled, idx


def kernel(input_mixed, ref_panel_0, ref_panel_1, ref_panel_2, weights):
    bs, n_snps = input_mixed.shape
    n_refs = ref_panel_0.shape[0]
    n_windows = n_snps // WIN
    p0, i0, p1, i1 = _tc_panels(
        input_mixed, ref_panel_0, ref_panel_1, weights, bs, n_refs, n_windows
    )
    p2, i2 = _sc_panel(input_mixed, ref_panel_2, weights, bs, n_windows)
    return (p0, p1, p2, i0, i1, i2)
